# baseline (speedup 1.0000x reference)
import functools

import jax
import jax.numpy as jnp
from jax import lax
from jax.experimental import pallas as pl
from jax.experimental.pallas import tpu as pltpu
from jax.experimental.pallas import tpu_sc as plsc

NC = 2
NS = 16
L = 16
NW = NC * NS

B = 16384
D = 128
CHUNK = 128
ROWS_PER_W = B // NW
NCHUNK = ROWS_PER_W // CHUNK
NSUB = D // L
NGROUP = CHUNK // L


def _gmf_body(vidx_hbm, hidx_hbm, virus_hbm, human_hbm, vb_hbm, hb_hbm,
              w_hbm, b_hbm, out_hbm,
              vidx_v, hidx_v, vrows_v, hrows_v, vb_v, hb_v, w_v, b_v,
              ws_v, pscr_v, out_v, sem, bsem):
    wid = lax.axis_index("s") * NC + lax.axis_index("c")

    lane_iota = lax.iota(jnp.int32, L)

    def hsplat(x):
        for k in (1, 2, 4, 8):
            idx = lax.bitwise_xor(lane_iota, k)
            x = x + x.at[idx].get(mode="promise_in_bounds")
        return x

    merge_masks = [(lane_iota & h) == 0 for h in (8, 4, 2, 1)]
    merge_perms = [lax.bitwise_xor(lane_iota, h) for h in (8, 4, 2, 1)]
    bitrev = [0, 8, 4, 12, 2, 10, 6, 14, 1, 9, 5, 13, 3, 11, 7, 15]

    def issue(j, s):
        pltpu.async_copy(virus_hbm.at[vidx_v.at[j, 0]], vrows_v.at[s],
                         sem.at[s])
        pltpu.async_copy(human_hbm.at[hidx_v.at[j, 0]], hrows_v.at[s],
                         sem.at[s])
        pltpu.async_copy(vb_hbm.at[vidx_v.at[j]], vb_v.at[s], bsem.at[s])
        pltpu.async_copy(hb_hbm.at[hidx_v.at[j]], hb_v.at[s], bsem.at[s])

    def chunk_body(j, carry):
        s = lax.rem(j, 2)
        ns = lax.rem(j + 1, 2)

        @pl.when(j == 0)
        def _():
            csem = sem.at[0]
            pltpu.async_copy(vidx_hbm.at[wid], vidx_v, csem)
            pltpu.async_copy(hidx_hbm.at[wid], hidx_v, csem)
            pltpu.async_copy(w_hbm, w_v, csem)
            pltpu.async_copy(b_hbm, b_v.at[pl.ds(0, 1)], csem)
            pltpu.make_async_copy(vidx_hbm.at[wid], vidx_v, csem).wait()
            pltpu.make_async_copy(hidx_hbm.at[wid], hidx_v, csem).wait()
            pltpu.make_async_copy(w_hbm, w_v, csem).wait()
            pltpu.make_async_copy(b_hbm, b_v.at[pl.ds(0, 1)], csem).wait()
            issue(0, 0)
            wtot = w_v[pl.ds(0, L)]
            for c in range(1, NSUB):
                wtot = wtot + w_v[pl.ds(c * L, L)]
            bvec0 = hsplat(jnp.where(lane_iota == 0, b_v[...], 0.0))
            ws_v[...] = hsplat(wtot)
            b_v[...] = bvec0

        @pl.when(j < NCHUNK - 1)
        def _():
            issue(j + 1, ns)

        pltpu.make_async_copy(virus_hbm.at[vidx_v.at[j, 0]], vrows_v.at[s],
                              sem.at[s]).wait()
        pltpu.make_async_copy(human_hbm.at[hidx_v.at[j, 0]], hrows_v.at[s],
                              sem.at[s]).wait()

        wvecs = [w_v[pl.ds(c * L, L)] for c in range(NSUB)]

        @plsc.parallel_loop(0, CHUNK, 1, unroll=8)
        def row_body(rr):
            acc0 = (vrows_v[s, rr, pl.ds(0, L)]
                    * hrows_v[s, rr, pl.ds(0, L)] * wvecs[0])
            acc1 = (vrows_v[s, rr, pl.ds(L, L)]
                    * hrows_v[s, rr, pl.ds(L, L)] * wvecs[1])
            for c in range(2, NSUB, 2):
                acc0 = acc0 + (vrows_v[s, rr, pl.ds(c * L, L)]
                               * hrows_v[s, rr, pl.ds(c * L, L)]
                               * wvecs[c])
                acc1 = acc1 + (vrows_v[s, rr, pl.ds((c + 1) * L, L)]
                               * hrows_v[s, rr, pl.ds((c + 1) * L, L)]
                               * wvecs[c + 1])
            pscr_v[rr, :] = acc0 + acc1

        pltpu.make_async_copy(vb_hbm.at[vidx_v.at[j]], vb_v.at[s],
                              bsem.at[s]).wait()
        pltpu.make_async_copy(hb_hbm.at[hidx_v.at[j]], hb_v.at[s],
                              bsem.at[s]).wait()

        @plsc.parallel_loop(0, NGROUP, 1, unroll=2)
        def group_body(g):
            rbase = g * L

            def tree(lo, span):
                if span == 1:
                    return pscr_v[rbase + bitrev[lo], :]
                half = span // 2
                x = tree(lo, half)
                y = tree(lo + half, half)
                stage = {8: 0, 4: 1, 2: 2, 1: 3}[L // span]
                prm = merge_perms[stage]
                u = x + x.at[prm].get(mode="promise_in_bounds")
                v = y + y.at[prm].get(mode="promise_in_bounds")
                return jnp.where(merge_masks[stage], u, v)

            dot = tree(0, L)
            vbg = vb_v[s, 0, pl.ds(rbase, L)]
            hbg = hb_v[s, 0, pl.ds(rbase, L)]
            z = dot + (vbg + hbg) * ws_v[...] + b_v[...]
            res = 1.0 / (1.0 + jnp.exp(-z))
            out_v[pl.ds(j * CHUNK + rbase, L)] = res

        return carry

    lax.fori_loop(0, NCHUNK, chunk_body, 0)
    pltpu.sync_copy(out_v, out_hbm.at[pl.ds(wid * ROWS_PER_W, ROWS_PER_W)])


@jax.jit
def _gmf(vidx, hidx, virus, human, vb, hb, w, b):
    mesh = plsc.VectorSubcoreMesh(core_axis_name="c", subcore_axis_name="s")
    run = functools.partial(
        pl.kernel,
        out_type=jax.ShapeDtypeStruct((B,), jnp.float32),
        mesh=mesh,
        scratch_types=[
            pltpu.VMEM((NCHUNK, 1, CHUNK), jnp.int32),
            pltpu.VMEM((NCHUNK, 1, CHUNK), jnp.int32),
            pltpu.VMEM((2, CHUNK, D), jnp.float32),
            pltpu.VMEM((2, CHUNK, D), jnp.float32),
            pltpu.VMEM((2, 1, CHUNK), jnp.float32),
            pltpu.VMEM((2, 1, CHUNK), jnp.float32),
            pltpu.VMEM((D,), jnp.float32),
            pltpu.VMEM((L,), jnp.float32),
            pltpu.VMEM((L,), jnp.float32),
            pltpu.VMEM((CHUNK, L), jnp.float32),
            pltpu.VMEM((ROWS_PER_W,), jnp.float32),
            pltpu.SemaphoreType.DMA((2,)),
            pltpu.SemaphoreType.DMA((2,)),
        ],
    )(_gmf_body)
    return run(vidx, hidx, virus, human, vb, hb, w, b)


def kernel(v_idxs, h_idxs, virus, human, vb, hb, W, b):
    vidx = v_idxs.astype(jnp.int32).reshape(NW, NCHUNK, 1, CHUNK)
    hidx = h_idxs.astype(jnp.int32).reshape(NW, NCHUNK, 1, CHUNK)
    out = _gmf(vidx, hidx, virus, human, vb.reshape(1, -1), hb.reshape(1, -1),
               W.reshape(-1), b)
    return out.reshape(B, 1)


# speedup vs baseline: 1.0006x; 1.0006x over previous
import functools

import jax
import jax.numpy as jnp
from jax import lax
from jax.experimental import pallas as pl
from jax.experimental.pallas import tpu as pltpu
from jax.experimental.pallas import tpu_sc as plsc

NC = 2
NS = 16
L = 16
NW = NC * NS

B = 16384
D = 128
CHUNK = 128
ROWS_PER_W = B // NW
NCHUNK = ROWS_PER_W // CHUNK
NSUB = D // L
NGROUP = CHUNK // L


def _gmf_body(vidx_hbm, hidx_hbm, virus_hbm, human_hbm, vb_hbm, hb_hbm,
              w_hbm, b_hbm, out_hbm,
              vidx_v, hidx_v, vrows_v, hrows_v, vb_v, hb_v, w_v, b_v,
              ws_v, pscr_v, out_v, sem, bsem):
    wid = lax.axis_index("s") * NC + lax.axis_index("c")

    lane_iota = lax.iota(jnp.int32, L)

    def hsplat(x):
        for k in (1, 2, 4, 8):
            idx = lax.bitwise_xor(lane_iota, k)
            x = x + x.at[idx].get(mode="promise_in_bounds")
        return x

    merge_masks = [(lane_iota & h) == 0 for h in (8, 4, 2, 1)]
    merge_perms = [lax.bitwise_xor(lane_iota, h) for h in (8, 4, 2, 1)]
    bitrev = [0, 8, 4, 12, 2, 10, 6, 14, 1, 9, 5, 13, 3, 11, 7, 15]

    def issue(j, s):
        pltpu.async_copy(virus_hbm.at[vidx_v.at[j, 0]], vrows_v.at[s],
                         sem.at[s])
        pltpu.async_copy(human_hbm.at[hidx_v.at[j, 0]], hrows_v.at[s],
                         sem.at[s])
        pltpu.async_copy(vb_hbm.at[vidx_v.at[j]], vb_v.at[s], bsem.at[s])
        pltpu.async_copy(hb_hbm.at[hidx_v.at[j]], hb_v.at[s], bsem.at[s])

    def chunk_body(j, carry):
        s = lax.rem(j, 3)
        ns = lax.rem(j + 1, 3)

        @pl.when(j == 0)
        def _():
            csem = sem.at[0]
            pltpu.async_copy(vidx_hbm.at[wid], vidx_v, csem)
            pltpu.async_copy(hidx_hbm.at[wid], hidx_v, csem)
            pltpu.async_copy(w_hbm, w_v, csem)
            pltpu.async_copy(b_hbm, b_v.at[pl.ds(0, 1)], csem)
            pltpu.make_async_copy(vidx_hbm.at[wid], vidx_v, csem).wait()
            pltpu.make_async_copy(hidx_hbm.at[wid], hidx_v, csem).wait()
            pltpu.make_async_copy(w_hbm, w_v, csem).wait()
            pltpu.make_async_copy(b_hbm, b_v.at[pl.ds(0, 1)], csem).wait()
            issue(0, 0)
            issue(1, 1)
            wtot = w_v[pl.ds(0, L)]
            for c in range(1, NSUB):
                wtot = wtot + w_v[pl.ds(c * L, L)]
            bvec0 = hsplat(jnp.where(lane_iota == 0, b_v[...], 0.0))
            ws_v[...] = hsplat(wtot)
            b_v[...] = bvec0

        @pl.when(jnp.logical_and(j > 0, j < NCHUNK - 1))
        def _():
            issue(j + 1, ns)

        pltpu.make_async_copy(virus_hbm.at[vidx_v.at[j, 0]], vrows_v.at[s],
                              sem.at[s]).wait()
        pltpu.make_async_copy(human_hbm.at[hidx_v.at[j, 0]], hrows_v.at[s],
                              sem.at[s]).wait()

        wvecs = [w_v[pl.ds(c * L, L)] for c in range(NSUB)]

        @plsc.parallel_loop(0, CHUNK, 1, unroll=8)
        def row_body(rr):
            acc0 = (vrows_v[s, rr, pl.ds(0, L)]
                    * hrows_v[s, rr, pl.ds(0, L)] * wvecs[0])
            acc1 = (vrows_v[s, rr, pl.ds(L, L)]
                    * hrows_v[s, rr, pl.ds(L, L)] * wvecs[1])
            for c in range(2, NSUB, 2):
                acc0 = acc0 + (vrows_v[s, rr, pl.ds(c * L, L)]
                               * hrows_v[s, rr, pl.ds(c * L, L)]
                               * wvecs[c])
                acc1 = acc1 + (vrows_v[s, rr, pl.ds((c + 1) * L, L)]
                               * hrows_v[s, rr, pl.ds((c + 1) * L, L)]
                               * wvecs[c + 1])
            pscr_v[rr, :] = acc0 + acc1

        pltpu.make_async_copy(vb_hbm.at[vidx_v.at[j]], vb_v.at[s],
                              bsem.at[s]).wait()
        pltpu.make_async_copy(hb_hbm.at[hidx_v.at[j]], hb_v.at[s],
                              bsem.at[s]).wait()

        @plsc.parallel_loop(0, NGROUP, 1, unroll=2)
        def group_body(g):
            rbase = g * L

            def tree(lo, span):
                if span == 1:
                    return pscr_v[rbase + bitrev[lo], :]
                half = span // 2
                x = tree(lo, half)
                y = tree(lo + half, half)
                stage = {8: 0, 4: 1, 2: 2, 1: 3}[L // span]
                prm = merge_perms[stage]
                u = x + x.at[prm].get(mode="promise_in_bounds")
                v = y + y.at[prm].get(mode="promise_in_bounds")
                return jnp.where(merge_masks[stage], u, v)

            dot = tree(0, L)
            vbg = vb_v[s, 0, pl.ds(rbase, L)]
            hbg = hb_v[s, 0, pl.ds(rbase, L)]
            z = dot + (vbg + hbg) * ws_v[...] + b_v[...]
            res = 1.0 / (1.0 + jnp.exp(-z))
            out_v[pl.ds(j * CHUNK + rbase, L)] = res

        return carry

    lax.fori_loop(0, NCHUNK, chunk_body, 0)
    pltpu.sync_copy(out_v, out_hbm.at[pl.ds(wid * ROWS_PER_W, ROWS_PER_W)])


@jax.jit
def _gmf(vidx, hidx, virus, human, vb, hb, w, b):
    mesh = plsc.VectorSubcoreMesh(core_axis_name="c", subcore_axis_name="s")
    run = functools.partial(
        pl.kernel,
        out_type=jax.ShapeDtypeStruct((B,), jnp.float32),
        mesh=mesh,
        scratch_types=[
            pltpu.VMEM((NCHUNK, 1, CHUNK), jnp.int32),
            pltpu.VMEM((NCHUNK, 1, CHUNK), jnp.int32),
            pltpu.VMEM((3, CHUNK, D), jnp.float32),
            pltpu.VMEM((3, CHUNK, D), jnp.float32),
            pltpu.VMEM((3, 1, CHUNK), jnp.float32),
            pltpu.VMEM((3, 1, CHUNK), jnp.float32),
            pltpu.VMEM((D,), jnp.float32),
            pltpu.VMEM((L,), jnp.float32),
            pltpu.VMEM((L,), jnp.float32),
            pltpu.VMEM((CHUNK, L), jnp.float32),
            pltpu.VMEM((ROWS_PER_W,), jnp.float32),
            pltpu.SemaphoreType.DMA((3,)),
            pltpu.SemaphoreType.DMA((3,)),
        ],
    )(_gmf_body)
    return run(vidx, hidx, virus, human, vb, hb, w, b)


def kernel(v_idxs, h_idxs, virus, human, vb, hb, W, b):
    vidx = v_idxs.astype(jnp.int32).reshape(NW, NCHUNK, 1, CHUNK)
    hidx = h_idxs.astype(jnp.int32).reshape(NW, NCHUNK, 1, CHUNK)
    out = _gmf(vidx, hidx, virus, human, vb.reshape(1, -1), hb.reshape(1, -1),
               W.reshape(-1), b)
    return out.reshape(B, 1)


# speedup vs baseline: 1.0050x; 1.0044x over previous
import functools

import jax
import jax.numpy as jnp
from jax import lax
from jax.experimental import pallas as pl
from jax.experimental.pallas import tpu as pltpu
from jax.experimental.pallas import tpu_sc as plsc

NC = 2
NS = 16
L = 16
NW = NC * NS

B = 16384
D = 128
CHUNK = 128
ROWS_PER_W = B // NW
NCHUNK = ROWS_PER_W // CHUNK
NSUB = D // L
NGROUP = CHUNK // L


def _gmf_body(vidx_hbm, hidx_hbm, virus_hbm, human_hbm, vb_hbm, hb_hbm,
              w_hbm, b_hbm, out_hbm,
              vidx_v, hidx_v, vrows_v, hrows_v, vb_v, hb_v, w_v, b_v,
              ws_v, pscr_v, out_v, sem, bsem):
    wid = lax.axis_index("s") * NC + lax.axis_index("c")

    lane_iota = lax.iota(jnp.int32, L)

    def hsplat(x):
        for k in (1, 2, 4, 8):
            idx = lax.bitwise_xor(lane_iota, k)
            x = x + x.at[idx].get(mode="promise_in_bounds")
        return x

    merge_masks = [(lane_iota & h) == 0 for h in (8, 4, 2, 1)]
    merge_perms = [lax.bitwise_xor(lane_iota, h) for h in (8, 4, 2, 1)]
    bitrev = [0, 8, 4, 12, 2, 10, 6, 14, 1, 9, 5, 13, 3, 11, 7, 15]

    def issue(j, s):
        pltpu.async_copy(virus_hbm.at[vidx_v.at[j, 0]], vrows_v.at[s],
                         sem.at[s])
        pltpu.async_copy(human_hbm.at[hidx_v.at[j, 0]], hrows_v.at[s],
                         sem.at[s])
        pltpu.async_copy(vb_hbm.at[vidx_v.at[j]], vb_v.at[s], bsem.at[s])
        pltpu.async_copy(hb_hbm.at[hidx_v.at[j]], hb_v.at[s], bsem.at[s])

    def chunk_body(j, carry):
        s = lax.rem(j, 2)
        ns = lax.rem(j + 1, 2)

        @pl.when(j == 0)
        def _():
            csem = sem.at[0]
            pltpu.async_copy(vidx_hbm.at[wid], vidx_v, csem)
            pltpu.async_copy(hidx_hbm.at[wid], hidx_v, csem)
            pltpu.async_copy(w_hbm, w_v, csem)
            pltpu.async_copy(b_hbm, b_v.at[pl.ds(0, 1)], csem)
            pltpu.make_async_copy(vidx_hbm.at[wid], vidx_v, csem).wait()
            pltpu.make_async_copy(hidx_hbm.at[wid], hidx_v, csem).wait()
            pltpu.make_async_copy(w_hbm, w_v, csem).wait()
            pltpu.make_async_copy(b_hbm, b_v.at[pl.ds(0, 1)], csem).wait()
            issue(0, 0)
            wtot = w_v[pl.ds(0, L)]
            for c in range(1, NSUB):
                wtot = wtot + w_v[pl.ds(c * L, L)]
            bvec0 = hsplat(jnp.where(lane_iota == 0, b_v[...], 0.0))
            ws_v[...] = hsplat(wtot)
            b_v[...] = bvec0

        @pl.when(j < NCHUNK - 1)
        def _():
            issue(j + 1, ns)

        pltpu.make_async_copy(virus_hbm.at[vidx_v.at[j, 0]], vrows_v.at[s],
                              sem.at[s]).wait()
        pltpu.make_async_copy(human_hbm.at[hidx_v.at[j, 0]], hrows_v.at[s],
                              sem.at[s]).wait()

        wvecs = [w_v[pl.ds(c * L, L)] for c in range(NSUB)]

        @plsc.parallel_loop(0, CHUNK, 1, unroll=8)
        def row_body(rr):
            acc0 = (vrows_v[s, rr, pl.ds(0, L)]
                    * hrows_v[s, rr, pl.ds(0, L)] * wvecs[0])
            acc1 = (vrows_v[s, rr, pl.ds(L, L)]
                    * hrows_v[s, rr, pl.ds(L, L)] * wvecs[1])
            for c in range(2, NSUB, 2):
                acc0 = acc0 + (vrows_v[s, rr, pl.ds(c * L, L)]
                               * hrows_v[s, rr, pl.ds(c * L, L)]
                               * wvecs[c])
                acc1 = acc1 + (vrows_v[s, rr, pl.ds((c + 1) * L, L)]
                               * hrows_v[s, rr, pl.ds((c + 1) * L, L)]
                               * wvecs[c + 1])
            pscr_v[rr, :] = acc0 + acc1

        pltpu.make_async_copy(vb_hbm.at[vidx_v.at[j]], vb_v.at[s],
                              bsem.at[s]).wait()
        pltpu.make_async_copy(hb_hbm.at[hidx_v.at[j]], hb_v.at[s],
                              bsem.at[s]).wait()

        @plsc.parallel_loop(0, NGROUP, 1, unroll=2)
        def group_body(g):
            rbase = g * L

            def tree(lo, span):
                if span == 1:
                    return pscr_v[rbase + bitrev[lo], :]
                half = span // 2
                x = tree(lo, half)
                y = tree(lo + half, half)
                stage = {8: 0, 4: 1, 2: 2, 1: 3}[L // span]
                prm = merge_perms[stage]
                u = x + x.at[prm].get(mode="promise_in_bounds")
                v = y + y.at[prm].get(mode="promise_in_bounds")
                return jnp.where(merge_masks[stage], u, v)

            dot = tree(0, L)
            vbg = vb_v[s, 0, pl.ds(rbase, L)]
            hbg = hb_v[s, 0, pl.ds(rbase, L)]
            z = dot + (vbg + hbg) * ws_v[...] + b_v[...]
            res = 1.0 / (1.0 + jnp.exp(-z))
            out_v[pl.ds(j * CHUNK + rbase, L)] = res

        return carry

    lax.fori_loop(0, NCHUNK, chunk_body, 0)
    pltpu.sync_copy(out_v, out_hbm.at[pl.ds(wid * ROWS_PER_W, ROWS_PER_W)])


@jax.jit
def _gmf(vidx, hidx, virus, human, vb, hb, w, b):
    mesh = plsc.VectorSubcoreMesh(core_axis_name="c", subcore_axis_name="s")
    run = functools.partial(
        pl.kernel,
        out_type=jax.ShapeDtypeStruct((B,), jnp.float32),
        mesh=mesh,
        scratch_types=[
            pltpu.VMEM((NCHUNK, 1, CHUNK), jnp.int32),
            pltpu.VMEM((NCHUNK, 1, CHUNK), jnp.int32),
            pltpu.VMEM((2, CHUNK, D), jnp.float32),
            pltpu.VMEM((2, CHUNK, D), jnp.float32),
            pltpu.VMEM((2, 1, CHUNK), jnp.float32),
            pltpu.VMEM((2, 1, CHUNK), jnp.float32),
            pltpu.VMEM((D,), jnp.float32),
            pltpu.VMEM((L,), jnp.float32),
            pltpu.VMEM((L,), jnp.float32),
            pltpu.VMEM((CHUNK, L), jnp.float32),
            pltpu.VMEM((ROWS_PER_W,), jnp.float32),
            pltpu.SemaphoreType.DMA((2,)),
            pltpu.SemaphoreType.DMA((2,)),
        ],
    )(_gmf_body)
    return run(vidx, hidx, virus, human, vb, hb, w, b)


def kernel(v_idxs, h_idxs, virus, human, vb, hb, W, b):
    vidx = v_idxs.astype(jnp.int32).reshape(NW, NCHUNK, 1, CHUNK)
    hidx = h_idxs.astype(jnp.int32).reshape(NW, NCHUNK, 1, CHUNK)
    out = _gmf(vidx, hidx, virus, human, vb.reshape(1, -1), hb.reshape(1, -1),
               W.reshape(-1), b)
    return out.reshape(B, 1)
